# baseline (speedup 1.0000x reference)
import functools

import jax
import jax.numpy as jnp
from jax import lax
from jax.experimental import pallas as pl
from jax.experimental.pallas import tpu as pltpu
from jax.experimental.pallas import tpu_sc as plsc

_CH = 80
_NT = 16
_L = 16
_G = 16
_CPT = 256


def _make_edge_kernel(N, EP, D, with_cnt):
    n_chunks = EP // _CH
    assert n_chunks == _CPT * _NT and D == 128
    NA = N + 128
    stripe_i = (NA // _NT) & ~7
    tail_i = NA - stripe_i * _NT
    stripe = (N // _NT) & ~7
    tail_o = N - stripe * _NT
    assert 0 <= tail_i <= _CH and 0 <= tail_o <= _CH and stripe >= _CH
    assert stripe_i >= stripe
    zcnt_sz = ((stripe_i + _L - 1) // _L) * _L

    mesh = plsc.VectorSubcoreMesh(core_axis_name="c", subcore_axis_name="s")

    out_type = [jax.ShapeDtypeStruct((N, D), jnp.float32),
                jax.ShapeDtypeStruct((N, D), jnp.float32)]
    if with_cnt:
        out_type.append(jax.ShapeDtypeStruct((N,), jnp.float32))

    scratch = [
        pltpu.VMEM((_G, _CH), jnp.int32),
        pltpu.VMEM((_G, _CH), jnp.int32),
        pltpu.VMEM((_G, _CH), jnp.float32),
        pltpu.VMEM((_CH, D), jnp.float32),
        pltpu.VMEM((_CH, D), jnp.float32),
        pltpu.VMEM((_CH, D), jnp.float32),
        pltpu.VMEM((_CH,), jnp.int32),
        pltpu.VMEM((_CH,), jnp.int32),
        pltpu.VMEM((_CH,), jnp.int32),
        pltpu.VMEM((_CH,), jnp.int32),
        pltpu.VMEM((_CH,), jnp.int32),
        pltpu.VMEM((_CH,), jnp.int32),
        pltpu.VMEM((_CH,), jnp.float32),
        pltpu.VMEM((zcnt_sz,), jnp.float32),
        pltpu.VMEM_SHARED((NA, D), jnp.float32),
        pltpu.VMEM_SHARED((NA,), jnp.float32),
    ] + [pltpu.SemaphoreType.DMA] * 6

    @functools.partial(pl.kernel, out_type=out_type, mesh=mesh,
                       scratch_types=scratch)
    def edge_kernel(x_hbm, src_hbm, dst_hbm, frac_hbm, *refs):
        if with_cnt:
            a_out, b_out, cnt_out = refs[:3]
            srefs = refs[3:]
        else:
            a_out, b_out = refs[:2]
            srefs = refs[2:]
        (srcT, dstT, fracT, rows0, rows1, rows2,
         srcv0, srcv1, srcv2, dstv0, dstv1, dstv2, onesv, zcnt,
         acc, acccnt, *sems_all) = srefs
        rows = (rows0, rows1, rows2)
        srcvs = (srcv0, srcv1, srcv2)
        dstvs = (dstv0, dstv1, dstv2)
        semg = sems_all[:3]
        sems = sems_all[3:]

        cid = lax.axis_index("c")
        sid = lax.axis_index("s")

        zero16 = jnp.zeros((_L,), jnp.float32)
        one16 = jnp.ones((_L,), jnp.float32)

        def zfill(j, _):
            for t in range(D // _L):
                rows0[j, pl.ds(t * _L, _L)] = zero16
            return 0
        lax.fori_loop(0, _CH, zfill, 0)
        for t in range(_CH // _L):
            onesv[pl.ds(t * _L, _L)] = one16
        for t in range(zcnt_sz // _L):
            zcnt[pl.ds(t * _L, _L)] = zero16

        base_i = sid * stripe_i
        nf_i = stripe_i // _CH
        rem_i = stripe_i - nf_i * _CH
        for k in range(nf_i):
            pltpu.sync_copy(rows0, acc.at[pl.ds(base_i + k * _CH, _CH)])
        if rem_i:
            pltpu.sync_copy(rows0.at[pl.ds(0, rem_i)],
                            acc.at[pl.ds(base_i + nf_i * _CH, rem_i)])
        if tail_i:
            @pl.when(sid == _NT - 1)
            def _():
                pltpu.sync_copy(rows0.at[pl.ds(0, tail_i)],
                                acc.at[pl.ds(stripe_i * _NT, tail_i)])
        if with_cnt:
            @pl.when(cid == 0)
            def _():
                pltpu.sync_copy(zcnt.at[pl.ds(0, stripe_i)],
                                acccnt.at[pl.ds(base_i, stripe_i)])
                if tail_i:
                    @pl.when(sid == _NT - 1)
                    def _():
                        pltpu.sync_copy(zcnt.at[pl.ds(0, tail_i)],
                                        acccnt.at[pl.ds(stripe_i * _NT, tail_i)])
        base = sid * stripe
        n_full = stripe // _CH
        rem = stripe - n_full * _CH
        plsc.subcore_barrier()

        def scale_rows(rb, k):
            def gbody(g, _):
                f16 = fracT[k, pl.ds(g * _L, _L)]
                for jj in range(_L):
                    j = g * _L + jj
                    f = f16[jj]
                    for t in range(D // _L):
                        sl = pl.ds(t * _L, _L)
                        rb[j, sl] = rb[j, sl] * f
                return 0
            lax.fori_loop(0, _CH // _L, gbody, 0)

        tbase = sid * _CPT

        def load_idx(dst_1d, src_2d, k):
            for t in range(_CH // _L):
                sl = pl.ds(t * _L, _L)
                dst_1d[sl] = src_2d[k, sl]

        def gbody(g, _):
            gbase = tbase + g * _G
            pltpu.sync_copy(src_hbm.at[pl.ds(gbase, _G)], srcT)
            pltpu.sync_copy(dst_hbm.at[pl.ds(gbase, _G)], dstT)

            @pl.when(cid == 1)
            def _():
                pltpu.sync_copy(frac_hbm.at[pl.ds(gbase, _G)], fracT)

            for p in range(2):
                load_idx(srcvs[p], srcT, p)
                pltpu.async_copy(x_hbm.at[srcvs[p]], rows[p], semg[p])
            for k in range(_G):
                b = k % 3
                pltpu.make_async_copy(x_hbm.at[srcvs[b]], rows[b],
                                      semg[b]).wait()

                @pl.when(cid == 1)
                def _():
                    scale_rows(rows[b], k)

                c = (k + 2) % 3
                if k >= 1:
                    pltpu.make_async_copy(rows[c], acc.at[dstvs[c]],
                                          sems[c]).wait()
                    if with_cnt:
                        @pl.when(cid == 0)
                        def _():
                            pltpu.make_async_copy(onesv, acccnt.at[dstvs[c]],
                                                  sems[c]).wait()
                if k + 2 < _G:
                    load_idx(srcvs[c], srcT, k + 2)
                    pltpu.async_copy(x_hbm.at[srcvs[c]], rows[c], semg[c])

                load_idx(dstvs[b], dstT, k)
                pltpu.async_copy(rows[b], acc.at[dstvs[b]], sems[b],
                                 add=True)
                if with_cnt:
                    @pl.when(cid == 0)
                    def _():
                        pltpu.async_copy(onesv, acccnt.at[dstvs[b]],
                                         sems[b], add=True)
            bl = (_G - 1) % 3
            pltpu.make_async_copy(rows[bl], acc.at[dstvs[bl]],
                                  sems[bl]).wait()
            if with_cnt:
                @pl.when(cid == 0)
                def _():
                    pltpu.make_async_copy(onesv, acccnt.at[dstvs[bl]],
                                          sems[bl]).wait()
            return 0
        lax.fori_loop(0, _CPT // _G, gbody, 0)

        plsc.subcore_barrier()

        def dump(src_ref, out):
            def blk(off, nrow):
                pltpu.sync_copy(src_ref.at[pl.ds(off, nrow)],
                                rows0.at[pl.ds(0, nrow)])
                pltpu.sync_copy(rows0.at[pl.ds(0, nrow)],
                                out.at[pl.ds(off, nrow)])
            for k in range(n_full):
                blk(base + k * _CH, _CH)
            if rem:
                blk(base + n_full * _CH, rem)
            if tail_o:
                @pl.when(sid == _NT - 1)
                def _():
                    blk(stripe * _NT, tail_o)

        @pl.when(cid == 0)
        def _():
            dump(acc, a_out)

        @pl.when(cid == 1)
        def _():
            dump(acc, b_out)

        if with_cnt:
            @pl.when(cid == 0)
            def _():
                pltpu.sync_copy(acccnt.at[pl.ds(base, stripe)],
                                zcnt.at[pl.ds(0, stripe)])
                pltpu.sync_copy(zcnt.at[pl.ds(0, stripe)],
                                cnt_out.at[pl.ds(base, stripe)])
                if tail_o:
                    @pl.when(sid == _NT - 1)
                    def _():
                        pltpu.sync_copy(acccnt.at[pl.ds(stripe * _NT, tail_o)],
                                        zcnt.at[pl.ds(0, tail_o)])
                        pltpu.sync_copy(zcnt.at[pl.ds(0, tail_o)],
                                        cnt_out.at[pl.ds(stripe * _NT, tail_o)])

    return edge_kernel


def _dense_body(a, b, xv, cv, w0, dw, rt, bs, o, *, act):
    f32 = jnp.float32
    acc = jnp.dot(a[...], w0[...], preferred_element_type=f32)
    acc += jnp.dot(b[...], dw[...], preferred_element_type=f32)
    acc = acc / jnp.maximum(cv[...], 1.0)
    acc = acc + jnp.dot(xv[...], rt[...], preferred_element_type=f32) + bs[...]
    if act == "relu":
        o[...] = jnp.maximum(acc, 0.0)
    else:
        m = jnp.max(acc, axis=1, keepdims=True)
        l = acc - m
        o[...] = l - jnp.log(jnp.sum(jnp.exp(l), axis=1, keepdims=True))


def _dense(a, b, x, cnt2d, w0, dw, root, bias2d, act):
    N, D = x.shape
    BN = 2000
    grid = (N // BN,)
    row_spec = lambda shp: pl.BlockSpec(shp, lambda i: (i, 0))
    w_spec = pl.BlockSpec((D, D), lambda i: (0, 0))
    return pl.pallas_call(
        functools.partial(_dense_body, act=act),
        grid=grid,
        in_specs=[row_spec((BN, D)), row_spec((BN, D)), row_spec((BN, D)),
                  row_spec((BN, 1)),
                  w_spec, w_spec, w_spec,
                  pl.BlockSpec((1, D), lambda i: (0, 0))],
        out_specs=row_spec((BN, D)),
        out_shape=jax.ShapeDtypeStruct((N, D), jnp.float32),
    )(a, b, x, cnt2d, w0, dw, root, bias2d)


def kernel(x, edge_index, edge_attr, W1, root1, b1, W2, root2, b2):
    N, D = x.shape
    E = edge_index.shape[1]
    src = edge_index[0]
    dst = edge_index[1]
    frac = edge_attr[:, 0]

    EP = _CPT * _NT * _CH
    pad = EP - E
    assert pad >= 0
    if pad:
        r = jnp.arange(pad, dtype=jnp.int32)
        src = jnp.concatenate([src, r % N])
        dst = jnp.concatenate([dst, N + (r % 128)])
        frac = jnp.concatenate([frac, jnp.zeros((pad,), jnp.float32)])
    src2d = src.reshape(-1, _CH)
    dst2d = dst.reshape(-1, _CH)
    frac2d = frac.reshape(-1, _CH)

    ek1 = _make_edge_kernel(N, EP, D, with_cnt=True)
    ek2 = _make_edge_kernel(N, EP, D, with_cnt=False)

    a, b, cnt = ek1(x, src2d, dst2d, frac2d)
    cnt2d = cnt.reshape(N, 1)
    h = _dense(a, b, x, cnt2d,
               W1[0], W1[1] - W1[0], root1, b1.reshape(1, D), "relu")
    a, b = ek2(h, src2d, dst2d, frac2d)
    return _dense(a, b, h, cnt2d,
                  W2[0], W2[1] - W2[0], root2, b2.reshape(1, D), "logsoftmax")


# speedup vs baseline: 1.0388x; 1.0388x over previous
import functools

import jax
import jax.numpy as jnp
from jax import lax
from jax.experimental import pallas as pl
from jax.experimental.pallas import tpu as pltpu
from jax.experimental.pallas import tpu_sc as plsc

_CH = 128
_NT = 16
_L = 16
_G = 16
_CPT = 160


def _make_edge_kernel(N, EP, D, with_cnt):
    n_chunks = EP // _CH
    assert n_chunks == _CPT * _NT and D == 128
    NA = N + 128
    stripe_i = (NA // _NT) & ~7
    tail_i = NA - stripe_i * _NT
    stripe = (N // _NT) & ~7
    tail_o = N - stripe * _NT
    assert 0 <= tail_i <= _CH and 0 <= tail_o <= _CH and stripe >= _CH
    assert stripe_i >= stripe
    zcnt_sz = ((stripe_i + _L - 1) // _L) * _L

    mesh = plsc.VectorSubcoreMesh(core_axis_name="c", subcore_axis_name="s")

    out_type = [jax.ShapeDtypeStruct((N, D), jnp.float32),
                jax.ShapeDtypeStruct((N, D), jnp.float32)]
    if with_cnt:
        out_type.append(jax.ShapeDtypeStruct((N,), jnp.float32))

    scratch = [
        pltpu.VMEM((_G, _CH), jnp.int32),
        pltpu.VMEM((_G, _CH), jnp.int32),
        pltpu.VMEM((_G, _CH), jnp.float32),
        pltpu.VMEM((_CH, D), jnp.float32),
        pltpu.VMEM((_CH, D), jnp.float32),
        pltpu.VMEM((_CH,), jnp.int32),
        pltpu.VMEM((_CH,), jnp.int32),
        pltpu.VMEM((_CH,), jnp.int32),
        pltpu.VMEM((_CH,), jnp.float32),
        pltpu.VMEM((zcnt_sz,), jnp.float32),
        pltpu.VMEM_SHARED((NA, D), jnp.float32),
        pltpu.VMEM_SHARED((NA,), jnp.float32),
    ] + [pltpu.SemaphoreType.DMA] * 2

    @functools.partial(pl.kernel, out_type=out_type, mesh=mesh,
                       scratch_types=scratch)
    def edge_kernel(x_hbm, src_hbm, dst_hbm, frac_hbm, *refs):
        if with_cnt:
            a_out, b_out, cnt_out = refs[:3]
            srefs = refs[3:]
        else:
            a_out, b_out = refs[:2]
            srefs = refs[2:]
        (srcT, dstT, fracT, rows0, rows1,
         srcv0, srcv1, dstv, onesv, zcnt,
         acc, acccnt, *sems_all) = srefs
        rows = (rows0, rows1)
        srcvs = (srcv0, srcv1)
        semg = sems_all

        cid = lax.axis_index("c")
        sid = lax.axis_index("s")

        zero16 = jnp.zeros((_L,), jnp.float32)
        one16 = jnp.ones((_L,), jnp.float32)

        def zfill(j, _):
            for t in range(D // _L):
                rows0[j, pl.ds(t * _L, _L)] = zero16
            return 0
        lax.fori_loop(0, _CH, zfill, 0)
        for t in range(_CH // _L):
            onesv[pl.ds(t * _L, _L)] = one16
        for t in range(zcnt_sz // _L):
            zcnt[pl.ds(t * _L, _L)] = zero16

        base_i = sid * stripe_i
        nf_i = stripe_i // _CH
        rem_i = stripe_i - nf_i * _CH
        for k in range(nf_i):
            pltpu.sync_copy(rows0, acc.at[pl.ds(base_i + k * _CH, _CH)])
        if rem_i:
            pltpu.sync_copy(rows0.at[pl.ds(0, rem_i)],
                            acc.at[pl.ds(base_i + nf_i * _CH, rem_i)])
        if tail_i:
            @pl.when(sid == _NT - 1)
            def _():
                pltpu.sync_copy(rows0.at[pl.ds(0, tail_i)],
                                acc.at[pl.ds(stripe_i * _NT, tail_i)])
        if with_cnt:
            @pl.when(cid == 0)
            def _():
                pltpu.sync_copy(zcnt.at[pl.ds(0, stripe_i)],
                                acccnt.at[pl.ds(base_i, stripe_i)])
                if tail_i:
                    @pl.when(sid == _NT - 1)
                    def _():
                        pltpu.sync_copy(zcnt.at[pl.ds(0, tail_i)],
                                        acccnt.at[pl.ds(stripe_i * _NT, tail_i)])
        base = sid * stripe
        n_full = stripe // _CH
        rem = stripe - n_full * _CH
        plsc.subcore_barrier()

        def scale_rows(rb, k):
            def gbody(g, _):
                f16 = fracT[k, pl.ds(g * _L, _L)]
                for jj in range(_L):
                    j = g * _L + jj
                    f = f16[jj]
                    for t in range(D // _L):
                        sl = pl.ds(t * _L, _L)
                        rb[j, sl] = rb[j, sl] * f
                return 0
            lax.fori_loop(0, _CH // _L, gbody, 0)

        tbase = sid * _CPT

        def load_idx(dst_1d, src_2d, k):
            for t in range(_CH // _L):
                sl = pl.ds(t * _L, _L)
                dst_1d[sl] = src_2d[k, sl]

        def gbody(g, _):
            gbase = tbase + g * _G
            pltpu.sync_copy(src_hbm.at[pl.ds(gbase, _G)], srcT)
            pltpu.sync_copy(dst_hbm.at[pl.ds(gbase, _G)], dstT)

            @pl.when(cid == 1)
            def _():
                pltpu.sync_copy(frac_hbm.at[pl.ds(gbase, _G)], fracT)

            load_idx(srcvs[0], srcT, 0)
            pltpu.async_copy(x_hbm.at[srcvs[0]], rows[0], semg[0])
            for k in range(_G):
                b = k % 2
                if k + 1 < _G:
                    load_idx(srcvs[1 - b], srcT, k + 1)
                    pltpu.async_copy(x_hbm.at[srcvs[1 - b]], rows[1 - b],
                                     semg[1 - b])
                pltpu.make_async_copy(x_hbm.at[srcvs[b]], rows[b],
                                      semg[b]).wait()

                @pl.when(cid == 1)
                def _():
                    scale_rows(rows[b], k)

                load_idx(dstv, dstT, k)
                pltpu.sync_copy(rows[b], acc.at[dstv], add=True)
                if with_cnt:
                    @pl.when(cid == 0)
                    def _():
                        pltpu.sync_copy(onesv, acccnt.at[dstv], add=True)
            return 0
        lax.fori_loop(0, _CPT // _G, gbody, 0)

        plsc.subcore_barrier()

        def dump(src_ref, out):
            def blk(off, nrow):
                pltpu.sync_copy(src_ref.at[pl.ds(off, nrow)],
                                rows0.at[pl.ds(0, nrow)])
                pltpu.sync_copy(rows0.at[pl.ds(0, nrow)],
                                out.at[pl.ds(off, nrow)])
            for k in range(n_full):
                blk(base + k * _CH, _CH)
            if rem:
                blk(base + n_full * _CH, rem)
            if tail_o:
                @pl.when(sid == _NT - 1)
                def _():
                    blk(stripe * _NT, tail_o)

        @pl.when(cid == 0)
        def _():
            dump(acc, a_out)

        @pl.when(cid == 1)
        def _():
            dump(acc, b_out)

        if with_cnt:
            @pl.when(cid == 0)
            def _():
                pltpu.sync_copy(acccnt.at[pl.ds(base, stripe)],
                                zcnt.at[pl.ds(0, stripe)])
                pltpu.sync_copy(zcnt.at[pl.ds(0, stripe)],
                                cnt_out.at[pl.ds(base, stripe)])
                if tail_o:
                    @pl.when(sid == _NT - 1)
                    def _():
                        pltpu.sync_copy(acccnt.at[pl.ds(stripe * _NT, tail_o)],
                                        zcnt.at[pl.ds(0, tail_o)])
                        pltpu.sync_copy(zcnt.at[pl.ds(0, tail_o)],
                                        cnt_out.at[pl.ds(stripe * _NT, tail_o)])

    return edge_kernel


def _dense_body(a, b, xv, cv, w0, dw, rt, bs, o, *, act):
    f32 = jnp.float32
    acc = jnp.dot(a[...], w0[...], preferred_element_type=f32)
    acc += jnp.dot(b[...], dw[...], preferred_element_type=f32)
    acc = acc / jnp.maximum(cv[...], 1.0)
    acc = acc + jnp.dot(xv[...], rt[...], preferred_element_type=f32) + bs[...]
    if act == "relu":
        o[...] = jnp.maximum(acc, 0.0)
    else:
        m = jnp.max(acc, axis=1, keepdims=True)
        l = acc - m
        o[...] = l - jnp.log(jnp.sum(jnp.exp(l), axis=1, keepdims=True))


def _dense(a, b, x, cnt2d, w0, dw, root, bias2d, act):
    N, D = x.shape
    BN = 2000
    grid = (N // BN,)
    row_spec = lambda shp: pl.BlockSpec(shp, lambda i: (i, 0))
    w_spec = pl.BlockSpec((D, D), lambda i: (0, 0))
    return pl.pallas_call(
        functools.partial(_dense_body, act=act),
        grid=grid,
        in_specs=[row_spec((BN, D)), row_spec((BN, D)), row_spec((BN, D)),
                  row_spec((BN, 1)),
                  w_spec, w_spec, w_spec,
                  pl.BlockSpec((1, D), lambda i: (0, 0))],
        out_specs=row_spec((BN, D)),
        out_shape=jax.ShapeDtypeStruct((N, D), jnp.float32),
    )(a, b, x, cnt2d, w0, dw, root, bias2d)


def kernel(x, edge_index, edge_attr, W1, root1, b1, W2, root2, b2):
    N, D = x.shape
    E = edge_index.shape[1]
    src = edge_index[0]
    dst = edge_index[1]
    frac = edge_attr[:, 0]

    EP = _CPT * _NT * _CH
    pad = EP - E
    assert pad >= 0
    if pad:
        r = jnp.arange(pad, dtype=jnp.int32)
        src = jnp.concatenate([src, r % N])
        dst = jnp.concatenate([dst, N + (r % 128)])
        frac = jnp.concatenate([frac, jnp.zeros((pad,), jnp.float32)])
    src2d = src.reshape(-1, _CH)
    dst2d = dst.reshape(-1, _CH)
    frac2d = frac.reshape(-1, _CH)

    ek1 = _make_edge_kernel(N, EP, D, with_cnt=True)

    a, b, cnt = ek1(x, src2d, dst2d, frac2d)
    cnt2d = cnt.reshape(N, 1)
    h = _dense(a, b, x, cnt2d,
               W1[0], W1[1] - W1[0], root1, b1.reshape(1, D), "relu")
    a, b, _ = ek1(h, src2d, dst2d, frac2d)
    return _dense(a, b, h, cnt2d,
                  W2[0], W2[1] - W2[0], root2, b2.reshape(1, D), "logsoftmax")


# speedup vs baseline: 1.0454x; 1.0063x over previous
import functools

import jax
import jax.numpy as jnp
from jax import lax
from jax.experimental import pallas as pl
from jax.experimental.pallas import tpu as pltpu
from jax.experimental.pallas import tpu_sc as plsc

_CH = 128
_NT = 16
_L = 16
_G = 16
_CPT = 160


def _make_edge_kernel(N, EP, D, with_cnt):
    n_chunks = EP // _CH
    assert n_chunks == _CPT * _NT and D == 128
    NA = N + 128
    stripe_i = (NA // _NT) & ~7
    tail_i = NA - stripe_i * _NT
    stripe = (N // _NT) & ~7
    tail_o = N - stripe * _NT
    assert 0 <= tail_i <= _CH and 0 <= tail_o <= _CH and stripe >= _CH
    assert stripe_i >= stripe
    zcnt_sz = ((stripe_i + _L - 1) // _L) * _L

    mesh = plsc.VectorSubcoreMesh(core_axis_name="c", subcore_axis_name="s")

    out_type = [jax.ShapeDtypeStruct((N, D), jnp.float32),
                jax.ShapeDtypeStruct((N, D), jnp.float32)]
    if with_cnt:
        out_type.append(jax.ShapeDtypeStruct((N,), jnp.float32))

    scratch = [
        pltpu.VMEM((_G, _CH), jnp.int32),
        pltpu.VMEM((_G, _CH), jnp.int32),
        pltpu.VMEM((_G, _CH), jnp.float32),
        pltpu.VMEM((_CH, D), jnp.float32),
        pltpu.VMEM((_CH, D), jnp.float32),
        pltpu.VMEM((_CH,), jnp.int32),
        pltpu.VMEM((_CH,), jnp.int32),
        pltpu.VMEM((_CH,), jnp.int32),
        pltpu.VMEM((_CH,), jnp.float32),
        pltpu.VMEM((zcnt_sz,), jnp.float32),
        pltpu.VMEM_SHARED((NA, D), jnp.float32),
        pltpu.VMEM_SHARED((NA,), jnp.float32),
    ] + [pltpu.SemaphoreType.DMA] * 2

    @functools.partial(pl.kernel, out_type=out_type, mesh=mesh,
                       scratch_types=scratch)
    def edge_kernel(x_hbm, src_hbm, dst_hbm, frac_hbm, *refs):
        if with_cnt:
            a_out, b_out, cnt_out = refs[:3]
            srefs = refs[3:]
        else:
            a_out, b_out = refs[:2]
            srefs = refs[2:]
        (srcT, dstT, fracT, rows0, rows1,
         srcv0, srcv1, dstv, onesv, zcnt,
         acc, acccnt, *sems_all) = srefs
        rows = (rows0, rows1)
        srcvs = (srcv0, srcv1)
        semg = sems_all

        cid = lax.axis_index("c")
        sid = lax.axis_index("s")

        zero16 = jnp.zeros((_L,), jnp.float32)
        one16 = jnp.ones((_L,), jnp.float32)

        def zfill(j, _):
            for t in range(D // _L):
                rows0[j, pl.ds(t * _L, _L)] = zero16
            return 0
        lax.fori_loop(0, _CH, zfill, 0)
        for t in range(_CH // _L):
            onesv[pl.ds(t * _L, _L)] = one16
        for t in range(zcnt_sz // _L):
            zcnt[pl.ds(t * _L, _L)] = zero16

        base_i = sid * stripe_i
        nf_i = stripe_i // _CH
        rem_i = stripe_i - nf_i * _CH
        for k in range(nf_i):
            pltpu.sync_copy(rows0, acc.at[pl.ds(base_i + k * _CH, _CH)])
        if rem_i:
            pltpu.sync_copy(rows0.at[pl.ds(0, rem_i)],
                            acc.at[pl.ds(base_i + nf_i * _CH, rem_i)])
        if tail_i:
            @pl.when(sid == _NT - 1)
            def _():
                pltpu.sync_copy(rows0.at[pl.ds(0, tail_i)],
                                acc.at[pl.ds(stripe_i * _NT, tail_i)])
        if with_cnt:
            @pl.when(cid == 0)
            def _():
                pltpu.sync_copy(zcnt.at[pl.ds(0, stripe_i)],
                                acccnt.at[pl.ds(base_i, stripe_i)])
                if tail_i:
                    @pl.when(sid == _NT - 1)
                    def _():
                        pltpu.sync_copy(zcnt.at[pl.ds(0, tail_i)],
                                        acccnt.at[pl.ds(stripe_i * _NT, tail_i)])
        base = sid * stripe
        n_full = stripe // _CH
        rem = stripe - n_full * _CH
        plsc.subcore_barrier()

        def scale_rows(rb, k):
            @plsc.parallel_loop(0, _CH // _L, unroll=2)
            def gbody(g):
                f16 = fracT[k, pl.ds(g * _L, _L)]
                for jj in range(_L):
                    j = g * _L + jj
                    f = f16[jj]
                    for t in range(D // _L):
                        sl = pl.ds(t * _L, _L)
                        rb[j, sl] = rb[j, sl] * f

        tbase = sid * _CPT

        def load_idx(dst_1d, src_2d, k):
            for t in range(_CH // _L):
                sl = pl.ds(t * _L, _L)
                dst_1d[sl] = src_2d[k, sl]

        def gbody(g, _):
            gbase = tbase + g * _G
            pltpu.sync_copy(src_hbm.at[pl.ds(gbase, _G)], srcT)
            pltpu.sync_copy(dst_hbm.at[pl.ds(gbase, _G)], dstT)

            @pl.when(cid == 1)
            def _():
                pltpu.sync_copy(frac_hbm.at[pl.ds(gbase, _G)], fracT)

            load_idx(srcvs[0], srcT, 0)
            pltpu.async_copy(x_hbm.at[srcvs[0]], rows[0], semg[0])
            for k in range(_G):
                b = k % 2
                if k + 1 < _G:
                    load_idx(srcvs[1 - b], srcT, k + 1)
                    pltpu.async_copy(x_hbm.at[srcvs[1 - b]], rows[1 - b],
                                     semg[1 - b])
                pltpu.make_async_copy(x_hbm.at[srcvs[b]], rows[b],
                                      semg[b]).wait()

                @pl.when(cid == 1)
                def _():
                    scale_rows(rows[b], k)

                load_idx(dstv, dstT, k)
                pltpu.sync_copy(rows[b], acc.at[dstv], add=True)
                if with_cnt:
                    @pl.when(cid == 0)
                    def _():
                        pltpu.sync_copy(onesv, acccnt.at[dstv], add=True)
            return 0
        lax.fori_loop(0, _CPT // _G, gbody, 0)

        plsc.subcore_barrier()

        def dump(src_ref, out):
            def blk(off, nrow):
                pltpu.sync_copy(src_ref.at[pl.ds(off, nrow)],
                                rows0.at[pl.ds(0, nrow)])
                pltpu.sync_copy(rows0.at[pl.ds(0, nrow)],
                                out.at[pl.ds(off, nrow)])
            for k in range(n_full):
                blk(base + k * _CH, _CH)
            if rem:
                blk(base + n_full * _CH, rem)
            if tail_o:
                @pl.when(sid == _NT - 1)
                def _():
                    blk(stripe * _NT, tail_o)

        @pl.when(cid == 0)
        def _():
            dump(acc, a_out)

        @pl.when(cid == 1)
        def _():
            dump(acc, b_out)

        if with_cnt:
            @pl.when(cid == 0)
            def _():
                pltpu.sync_copy(acccnt.at[pl.ds(base, stripe)],
                                zcnt.at[pl.ds(0, stripe)])
                pltpu.sync_copy(zcnt.at[pl.ds(0, stripe)],
                                cnt_out.at[pl.ds(base, stripe)])
                if tail_o:
                    @pl.when(sid == _NT - 1)
                    def _():
                        pltpu.sync_copy(acccnt.at[pl.ds(stripe * _NT, tail_o)],
                                        zcnt.at[pl.ds(0, tail_o)])
                        pltpu.sync_copy(zcnt.at[pl.ds(0, tail_o)],
                                        cnt_out.at[pl.ds(stripe * _NT, tail_o)])

    return edge_kernel


def _dense_body(a, b, xv, cv, w0, dw, rt, bs, o, *, act):
    f32 = jnp.float32
    acc = jnp.dot(a[...], w0[...], preferred_element_type=f32)
    acc += jnp.dot(b[...], dw[...], preferred_element_type=f32)
    acc = acc / jnp.maximum(cv[...], 1.0)
    acc = acc + jnp.dot(xv[...], rt[...], preferred_element_type=f32) + bs[...]
    if act == "relu":
        o[...] = jnp.maximum(acc, 0.0)
    else:
        m = jnp.max(acc, axis=1, keepdims=True)
        l = acc - m
        o[...] = l - jnp.log(jnp.sum(jnp.exp(l), axis=1, keepdims=True))


def _dense(a, b, x, cnt2d, w0, dw, root, bias2d, act):
    N, D = x.shape
    BN = 2000
    grid = (N // BN,)
    row_spec = lambda shp: pl.BlockSpec(shp, lambda i: (i, 0))
    w_spec = pl.BlockSpec((D, D), lambda i: (0, 0))
    return pl.pallas_call(
        functools.partial(_dense_body, act=act),
        grid=grid,
        in_specs=[row_spec((BN, D)), row_spec((BN, D)), row_spec((BN, D)),
                  row_spec((BN, 1)),
                  w_spec, w_spec, w_spec,
                  pl.BlockSpec((1, D), lambda i: (0, 0))],
        out_specs=row_spec((BN, D)),
        out_shape=jax.ShapeDtypeStruct((N, D), jnp.float32),
    )(a, b, x, cnt2d, w0, dw, root, bias2d)


def kernel(x, edge_index, edge_attr, W1, root1, b1, W2, root2, b2):
    N, D = x.shape
    E = edge_index.shape[1]
    src = edge_index[0]
    dst = edge_index[1]
    frac = edge_attr[:, 0]

    EP = _CPT * _NT * _CH
    pad = EP - E
    assert pad >= 0
    if pad:
        r = jnp.arange(pad, dtype=jnp.int32)
        src = jnp.concatenate([src, r % N])
        dst = jnp.concatenate([dst, N + (r % 128)])
        frac = jnp.concatenate([frac, jnp.zeros((pad,), jnp.float32)])
    src2d = src.reshape(-1, _CH)
    dst2d = dst.reshape(-1, _CH)
    frac2d = frac.reshape(-1, _CH)

    ek1 = _make_edge_kernel(N, EP, D, with_cnt=True)

    a, b, cnt = ek1(x, src2d, dst2d, frac2d)
    cnt2d = cnt.reshape(N, 1)
    h = _dense(a, b, x, cnt2d,
               W1[0], W1[1] - W1[0], root1, b1.reshape(1, D), "relu")
    a, b, _ = ek1(h, src2d, dst2d, frac2d)
    return _dense(a, b, h, cnt2d,
                  W2[0], W2[1] - W2[0], root2, b2.reshape(1, D), "logsoftmax")
